# baseline (speedup 1.0000x reference)
import jax
import jax.numpy as jnp
from jax.experimental import pallas as pl

B = 16384
COND = 128
K = 10
BI = 5.0
BB = 256

V_B1T, V_B2T, V_B34T, V_B5T, V_KEEP, V_ADD = 0, 1, 2, 3, 4, 5
V_R0_1, V_C0_1, V_C1_1, V_C2_1 = 6, 7, 8, 9
V_R0_2, V_C0_2, V_C1_2, V_C2_2 = 10, 11, 12, 13
V_DBASE = 14
NVEC = 15

M_BD2, M_BD34, M_BD5 = 0, 1, 2
M_A0_1, M_A1_1, M_A2_1 = 3, 4, 5
M_A0_2, M_A1_2, M_A2_2 = 6, 7, 8
M_CW, M_CH, M_PD = 9, 10, 11
NMAT = 12


def _dot(a, b):
    return jnp.dot(a, b, preferred_element_type=jnp.float32,
                   precision=jax.lax.Precision.HIGHEST)


def _spline(raw, xcol, mats, dbase, li):
    liw = (li < K).astype(jnp.float32)
    lih = ((li >= K) & (li < 2 * K)).astype(jnp.float32)
    lid = ((li >= 2 * K) & (li < 3 * K - 1)).astype(jnp.float32)

    neg = jnp.float32(-1e30)
    mw = jnp.max(jnp.where(liw > 0, raw, neg), axis=1, keepdims=True)
    mh = jnp.max(jnp.where(lih > 0, raw, neg), axis=1, keepdims=True)
    ew = jnp.where(liw > 0, jnp.exp(raw - mw), 0.0)
    eh = jnp.where(lih > 0, jnp.exp(raw - mh), 0.0)
    sw = jnp.sum(ew, axis=1, keepdims=True)
    sh = jnp.sum(eh, axis=1, keepdims=True)
    cw = _dot(ew, mats[M_CW])
    ch = _dot(eh, mats[M_CH])
    w = cw * (2.0 * BI / sw) - BI
    h = ch * (2.0 * BI / sh) - BI
    sp = jnp.where(lid > 0, jax.nn.softplus(raw) + 0.001, 0.0)
    d = _dot(sp, mats[M_PD]) + dbase

    cnt = jnp.sum(
        jnp.where((li <= K).astype(jnp.float32) * (w <= xcol) > 0, 1.0, 0.0),
        axis=1, keepdims=True)
    idx = jnp.clip(cnt, 1.0, float(K))
    lif = li.astype(jnp.float32)
    ohl = jnp.where(lif == idx - 1.0, 1.0, 0.0)
    ohr = jnp.where(lif == idx, 1.0, 0.0)

    def pick(arr, oh):
        return jnp.sum(arr * oh, axis=1, keepdims=True)

    wl, wr = pick(w, ohl), pick(w, ohr)
    hl, hr = pick(h, ohl), pick(h, ohr)
    dl, dr = pick(d, ohl), pick(d, ohr)

    tau = (xcol - wl) / (wr - wl)
    sk = (hr - hl) / (wr - wl)
    omt = 1.0 - tau
    v1 = sk * tau * tau + dl * tau * omt
    v2 = sk + (dr + dl - 2.0 * sk) * tau * omt
    y = hl + v1 / v2 * (hr - hl)
    d1 = sk * sk * (dr * tau * tau + 2.0 * sk * tau * omt + dl * omt * omt)
    pd = d1 / (v2 * v2)
    jac = jnp.abs(pd)
    return y, jac


def _g_kernel(c4_ref, pw1_ref, pt6_ref, vec_ref, mat_ref, g4_ref):
    vec = vec_ref[...]
    mats = mat_ref

    def vrow(i):
        return vec[i:i + 1, :]

    c4 = c4_ref[...]
    h1 = jnp.maximum(_dot(c4, pw1_ref[...]) + vrow(V_B1T), 0.0)
    h2 = jnp.maximum(_dot(h1, mats[M_BD2]) + vrow(V_B2T), 0.0)
    m1 = jnp.maximum(_dot(h2, mats[M_BD34]) + vrow(V_B34T), 0.0)
    m2 = jnp.maximum(_dot(m1, mats[M_BD5]) + vrow(V_B5T), 0.0)
    g4_ref[...] = _dot(m2, pt6_ref[...])


def _main_kernel(x_ref, g0_ref, vec_ref, mat_ref, y_ref, j_ref):
    vec = vec_ref[...]
    mats = mat_ref
    li = jax.lax.broadcasted_iota(jnp.int32, (1, 128), 1)

    def vrow(i):
        return vec[i:i + 1, :]

    g = g0_ref[...] * vrow(V_KEEP) + vrow(V_ADD)

    x1 = x_ref[:, 0:1]
    x2 = x_ref[:, 1:2]

    def fstage(col, a0, a1, a2, r0, c0, c1, c2):
        hh = jnp.maximum(_dot(g, mats[a0]) + col * vrow(r0) + vrow(c0), 0.0)
        hh = jnp.maximum(_dot(hh, mats[a1]) + vrow(c1), 0.0)
        return _dot(hh, mats[a2]) + vrow(c2)

    dbase = vrow(V_DBASE)
    raw2 = fstage(x1, M_A0_1, M_A1_1, M_A2_1, V_R0_1, V_C0_1, V_C1_1, V_C2_1)
    y2, jac2 = _spline(raw2, x2, mats, dbase, li)
    raw1 = fstage(y2, M_A0_2, M_A1_2, M_A2_2, V_R0_2, V_C0_2, V_C1_2, V_C2_2)
    y1, jac1 = _spline(raw1, x1, mats, dbase, li)

    y_ref[...] = jnp.concatenate([y1, y2], axis=1)
    j_ref[...] = jac1 * jac2


def _pad(a, shape):
    out = jnp.zeros(shape, jnp.float32)
    return out.at[tuple(slice(0, s) for s in a.shape)].set(a)


def kernel(x, c, params, index_p, index_v):
    f32 = jnp.float32
    (W1, b1), (W2, b2), (W3, b3) = params['fc_add']
    (W4, b4), (W5, b5), (W6, b6) = params['fc_min']
    null = params['null_token']
    num_nodes = COND // 2 + 1

    W34 = W3 @ W4
    b34 = b3 @ W4 + b4
    eye4 = jnp.eye(4, dtype=f32)
    pw1 = jnp.kron(eye4, W1)
    pt6 = jnp.kron(eye4, W6)
    bd2 = jnp.kron(eye4, W2)
    bd34 = jnp.kron(eye4, W34)
    bd5 = jnp.kron(eye4, W5)

    def tile4(v):
        return jnp.tile(v.reshape(1, 32), (1, 4)).reshape(128)

    nm1 = jax.nn.relu(null @ W4 + b4)
    nm2 = jax.nn.relu(nm1 @ W5 + b5)
    n0 = (nm2 @ W6 + b6)[0, 0]
    sv = jnp.sin(jnp.asarray(index_v, f32))
    gh1 = jax.nn.relu(sv * W1[0] + b1)
    gh2 = jax.nn.relu(gh1 @ W2 + b2)
    ga = gh2 @ W3 + b3
    gm1 = jax.nn.relu(ga @ W4 + b4)
    gm2 = jax.nn.relu(gm1 @ W5 + b5)
    gv = (gm2 @ W6 + b6)[0]

    ip = jnp.asarray(index_p, jnp.int32)
    keepv = jnp.ones((128,), f32).at[ip].set(0.0).at[ip + num_nodes - 1].set(0.0)
    addv = jnp.zeros((128,), f32).at[ip].set(n0).at[ip + num_nodes - 1].set(n0)
    addv = addv + b6[0] * keepv

    def fprep(layers):
        (U0, c0), (U1, c1), (U2, c2) = layers
        a0 = _pad(U0[1:1 + COND], (128, 128))
        r0 = _pad(U0[0:1], (1, 128))[0]
        cc0 = _pad((gv * U0[COND + 1] + c0)[None, :], (1, 128))[0]
        a1 = _pad(U1, (128, 128))
        cc1 = _pad(c1[None, :], (1, 128))[0]
        a2 = _pad(U2, (128, 128))
        cc2 = _pad(c2[None, :], (1, 128))[0]
        return a0, a1, a2, r0, cc0, cc1, cc2

    a0_1, a1_1, a2_1, r0_1, c0_1, c1_1, c2_1 = fprep(params['f1'])
    a0_2, a1_2, a2_2, r0_2, c0_2, c1_2, c2_2 = fprep(params['f2'])

    ii = jnp.arange(128)
    jj = jnp.arange(128)
    I, J = jnp.meshgrid(ii, jj, indexing='ij')
    cwm = ((I < K) & (J >= 1) & (J <= K) & (I <= J - 1)).astype(f32)
    chm = ((I >= K) & (I < 2 * K) & (J >= 1) & (J <= K)
           & (I - K <= J - 1)).astype(f32)
    pdm = ((I >= 2 * K) & (I < 3 * K - 1) & (J == I - 2 * K + 1)).astype(f32)
    dbase = jnp.zeros((128,), f32).at[0].set(1.0).at[K].set(1.0)

    vecs = jnp.stack([
        tile4(b1), tile4(b2), tile4(b34), tile4(b5), keepv, addv,
        r0_1, c0_1, c1_1, c2_1, r0_2, c0_2, c1_2, c2_2, dbase,
    ])
    matsarr = jnp.stack([
        bd2, bd34, bd5, a0_1, a1_1, a2_1, a0_2, a1_2, a2_2, cwm, chm, pdm,
    ])

    c4 = c.reshape(B * 32, 4)

    rep = lambda i: (0, 0)
    rep3 = lambda i: (0, 0, 0)
    g4 = pl.pallas_call(
        _g_kernel,
        grid=(B // BB,),
        in_specs=[
            pl.BlockSpec((BB * 32, 4), lambda i: (i, 0)),
            pl.BlockSpec((4, 128), rep),
            pl.BlockSpec((128, 4), rep),
            pl.BlockSpec((NVEC, 128), rep),
            pl.BlockSpec((NMAT, 128, 128), rep3),
        ],
        out_specs=pl.BlockSpec((BB * 32, 4), lambda i: (i, 0)),
        out_shape=jax.ShapeDtypeStruct((B * 32, 4), f32),
    )(c4, pw1, pt6, vecs, matsarr)
    g0 = g4.reshape(B, 128)

    y, jac = pl.pallas_call(
        _main_kernel,
        grid=(B // BB,),
        in_specs=[
            pl.BlockSpec((BB, 2), lambda i: (i, 0)),
            pl.BlockSpec((BB, 128), lambda i: (i, 0)),
            pl.BlockSpec((NVEC, 128), rep),
            pl.BlockSpec((NMAT, 128, 128), rep3),
        ],
        out_specs=[
            pl.BlockSpec((BB, 2), lambda i: (i, 0)),
            pl.BlockSpec((BB, 1), lambda i: (i, 0)),
        ],
        out_shape=[
            jax.ShapeDtypeStruct((B, 2), f32),
            jax.ShapeDtypeStruct((B, 1), f32),
        ],
    )(x, g0, vecs, matsarr)
    return y, jac


# speedup vs baseline: 2.0799x; 2.0799x over previous
import jax
import jax.numpy as jnp
from jax.experimental import pallas as pl

B = 16384
COND = 128
K = 10
BI = 5.0
BB = 256

V_B1T, V_B2T, V_B34T, V_B5T, V_KEEP, V_ADD = 0, 1, 2, 3, 4, 5
V_R0_1, V_C0_1, V_C1_1, V_C2_1 = 6, 7, 8, 9
V_R0_2, V_C0_2, V_C1_2, V_C2_2 = 10, 11, 12, 13
V_DBASE = 14
NVEC = 15

M_BD2, M_BD34, M_BD5 = 0, 1, 2
M_A0_1, M_A1_1, M_A2_1 = 3, 4, 5
M_A0_2, M_A1_2, M_A2_2 = 6, 7, 8
M_CW, M_CH, M_PD = 9, 10, 11
NMAT = 12


def _bdot(a, b):
    return jnp.dot(a, b, preferred_element_type=jnp.float32)


def _split(a):
    ahi = a.astype(jnp.bfloat16)
    alo = (a - ahi.astype(jnp.float32)).astype(jnp.bfloat16)
    return ahi, alo


def _dot3(a, w3):
    ahi, alo = _split(a)
    a3 = jnp.concatenate([ahi, alo, ahi], axis=1)
    return _bdot(a3, w3)


def _spline(raw, xcol, m3, dbase, li):
    liw = (li < K).astype(jnp.float32)
    lih = ((li >= K) & (li < 2 * K)).astype(jnp.float32)
    lid = ((li >= 2 * K) & (li < 3 * K - 1)).astype(jnp.float32)

    neg = jnp.float32(-1e30)
    mw = jnp.max(jnp.where(liw > 0, raw, neg), axis=1, keepdims=True)
    mh = jnp.max(jnp.where(lih > 0, raw, neg), axis=1, keepdims=True)
    ew = jnp.where(liw > 0, jnp.exp(raw - mw), 0.0)
    eh = jnp.where(lih > 0, jnp.exp(raw - mh), 0.0)
    sw = jnp.sum(ew, axis=1, keepdims=True)
    sh = jnp.sum(eh, axis=1, keepdims=True)
    cw = _dot3(ew, m3[M_CW])
    ch = _dot3(eh, m3[M_CH])
    w = cw * (2.0 * BI / sw) - BI
    h = ch * (2.0 * BI / sh) - BI
    sp = jnp.where(lid > 0, jax.nn.softplus(raw) + 0.001, 0.0)
    d = _dot3(sp, m3[M_PD]) + dbase

    cnt = jnp.sum(
        jnp.where((li <= K).astype(jnp.float32) * (w <= xcol) > 0, 1.0, 0.0),
        axis=1, keepdims=True)
    idx = jnp.clip(cnt, 1.0, float(K))
    lif = li.astype(jnp.float32)
    ohl = jnp.where(lif == idx - 1.0, 1.0, 0.0)
    ohr = jnp.where(lif == idx, 1.0, 0.0)

    def pick(arr, oh):
        return jnp.sum(arr * oh, axis=1, keepdims=True)

    wl, wr = pick(w, ohl), pick(w, ohr)
    hl, hr = pick(h, ohl), pick(h, ohr)
    dl, dr = pick(d, ohl), pick(d, ohr)

    tau = (xcol - wl) / (wr - wl)
    sk = (hr - hl) / (wr - wl)
    omt = 1.0 - tau
    v1 = sk * tau * tau + dl * tau * omt
    v2 = sk + (dr + dl - 2.0 * sk) * tau * omt
    y = hl + v1 / v2 * (hr - hl)
    d1 = sk * sk * (dr * tau * tau + 2.0 * sk * tau * omt + dl * omt * omt)
    pd = d1 / (v2 * v2)
    jac = jnp.abs(pd)
    return y, jac


def _g_kernel(c4_ref, pw1_ref, pt6_ref, vec_ref, m3_ref, g4_ref):
    vec = vec_ref[...]
    m3 = m3_ref

    def vrow(i):
        return vec[i:i + 1, :]

    c4 = c4_ref[...]
    h1 = jnp.maximum(_dot3(c4, pw1_ref[...]) + vrow(V_B1T), 0.0)
    h2 = jnp.maximum(_dot3(h1, m3[M_BD2]) + vrow(V_B2T), 0.0)
    m1 = jnp.maximum(_dot3(h2, m3[M_BD34]) + vrow(V_B34T), 0.0)
    m2 = jnp.maximum(_dot3(m1, m3[M_BD5]) + vrow(V_B5T), 0.0)
    g4_ref[...] = _dot3(m2, pt6_ref[...])


def _main_kernel(x_ref, g0_ref, vec_ref, m3_ref, y_ref, j_ref):
    vec = vec_ref[...]
    m3 = m3_ref
    li = jax.lax.broadcasted_iota(jnp.int32, (1, 128), 1)

    def vrow(i):
        return vec[i:i + 1, :]

    g = g0_ref[...] * vrow(V_KEEP) + vrow(V_ADD)

    x1 = x_ref[:, 0:1]
    x2 = x_ref[:, 1:2]

    def fstage(col, a0, a1, a2, r0, c0, c1, c2):
        hh = jnp.maximum(_dot3(g, m3[a0]) + col * vrow(r0) + vrow(c0), 0.0)
        hh = jnp.maximum(_dot3(hh, m3[a1]) + vrow(c1), 0.0)
        return _dot3(hh, m3[a2]) + vrow(c2)

    dbase = vrow(V_DBASE)
    raw2 = fstage(x1, M_A0_1, M_A1_1, M_A2_1, V_R0_1, V_C0_1, V_C1_1, V_C2_1)
    y2, jac2 = _spline(raw2, x2, m3, dbase, li)
    raw1 = fstage(y2, M_A0_2, M_A1_2, M_A2_2, V_R0_2, V_C0_2, V_C1_2, V_C2_2)
    y1, jac1 = _spline(raw1, x1, m3, dbase, li)

    y_ref[...] = jnp.concatenate([y1, y2], axis=1)
    j_ref[...] = jac1 * jac2


def _pad(a, shape):
    out = jnp.zeros(shape, jnp.float32)
    return out.at[tuple(slice(0, s) for s in a.shape)].set(a)


def kernel(x, c, params, index_p, index_v):
    f32 = jnp.float32
    (W1, b1), (W2, b2), (W3, b3) = params['fc_add']
    (W4, b4), (W5, b5), (W6, b6) = params['fc_min']
    null = params['null_token']
    num_nodes = COND // 2 + 1

    W34 = W3 @ W4
    b34 = b3 @ W4 + b4
    eye4 = jnp.eye(4, dtype=f32)
    pw1 = jnp.kron(eye4, W1)
    pt6 = jnp.kron(eye4, W6)
    bd2 = jnp.kron(eye4, W2)
    bd34 = jnp.kron(eye4, W34)
    bd5 = jnp.kron(eye4, W5)

    def tile4(v):
        return jnp.tile(v.reshape(1, 32), (1, 4)).reshape(128)

    nm1 = jax.nn.relu(null @ W4 + b4)
    nm2 = jax.nn.relu(nm1 @ W5 + b5)
    n0 = (nm2 @ W6 + b6)[0, 0]
    sv = jnp.sin(jnp.asarray(index_v, f32))
    gh1 = jax.nn.relu(sv * W1[0] + b1)
    gh2 = jax.nn.relu(gh1 @ W2 + b2)
    ga = gh2 @ W3 + b3
    gm1 = jax.nn.relu(ga @ W4 + b4)
    gm2 = jax.nn.relu(gm1 @ W5 + b5)
    gv = (gm2 @ W6 + b6)[0]

    ip = jnp.asarray(index_p, jnp.int32)
    keepv = jnp.ones((128,), f32).at[ip].set(0.0).at[ip + num_nodes - 1].set(0.0)
    addv = jnp.zeros((128,), f32).at[ip].set(n0).at[ip + num_nodes - 1].set(n0)
    addv = addv + b6[0] * keepv

    def fprep(layers):
        (U0, c0), (U1, c1), (U2, c2) = layers
        a0 = _pad(U0[1:1 + COND], (128, 128))
        r0 = _pad(U0[0:1], (1, 128))[0]
        cc0 = _pad((gv * U0[COND + 1] + c0)[None, :], (1, 128))[0]
        a1 = _pad(U1, (128, 128))
        cc1 = _pad(c1[None, :], (1, 128))[0]
        a2 = _pad(U2, (128, 128))
        cc2 = _pad(c2[None, :], (1, 128))[0]
        return a0, a1, a2, r0, cc0, cc1, cc2

    a0_1, a1_1, a2_1, r0_1, c0_1, c1_1, c2_1 = fprep(params['f1'])
    a0_2, a1_2, a2_2, r0_2, c0_2, c1_2, c2_2 = fprep(params['f2'])

    ii = jnp.arange(128)
    jj = jnp.arange(128)
    I, J = jnp.meshgrid(ii, jj, indexing='ij')
    cwm = ((I < K) & (J >= 1) & (J <= K) & (I <= J - 1)).astype(f32)
    chm = ((I >= K) & (I < 2 * K) & (J >= 1) & (J <= K)
           & (I - K <= J - 1)).astype(f32)
    pdm = ((I >= 2 * K) & (I < 3 * K - 1) & (J == I - 2 * K + 1)).astype(f32)
    dbase = jnp.zeros((128,), f32).at[0].set(1.0).at[K].set(1.0)

    vecs = jnp.stack([
        tile4(b1), tile4(b2), tile4(b34), tile4(b5), keepv, addv,
        r0_1, c0_1, c1_1, c2_1, r0_2, c0_2, c1_2, c2_2, dbase,
    ])
    matsarr = jnp.stack([
        bd2, bd34, bd5, a0_1, a1_1, a2_1, a0_2, a1_2, a2_2, cwm, chm, pdm,
    ])

    bf16 = jnp.bfloat16

    def w3(m):
        hi = m.astype(bf16)
        lo = (m - hi.astype(f32)).astype(bf16)
        return jnp.concatenate([hi, hi, lo], axis=0)

    mats3 = jax.vmap(w3)(matsarr)
    pw1s = w3(pw1)
    pt6s = w3(pt6)

    c4 = c.reshape(B * 32, 4)

    rep = lambda i: (0, 0)
    rep3 = lambda i: (0, 0, 0)
    g4 = pl.pallas_call(
        _g_kernel,
        grid=(B // BB,),
        in_specs=[
            pl.BlockSpec((BB * 32, 4), lambda i: (i, 0)),
            pl.BlockSpec((12, 128), rep),
            pl.BlockSpec((384, 4), rep),
            pl.BlockSpec((NVEC, 128), rep),
            pl.BlockSpec((NMAT, 384, 128), rep3),
        ],
        out_specs=pl.BlockSpec((BB * 32, 4), lambda i: (i, 0)),
        out_shape=jax.ShapeDtypeStruct((B * 32, 4), f32),
    )(c4, pw1s, pt6s, vecs, mats3)
    g0 = g4.reshape(B, 128)

    y, jac = pl.pallas_call(
        _main_kernel,
        grid=(B // BB,),
        in_specs=[
            pl.BlockSpec((BB, 2), lambda i: (i, 0)),
            pl.BlockSpec((BB, 128), lambda i: (i, 0)),
            pl.BlockSpec((NVEC, 128), rep),
            pl.BlockSpec((NMAT, 384, 128), rep3),
        ],
        out_specs=[
            pl.BlockSpec((BB, 2), lambda i: (i, 0)),
            pl.BlockSpec((BB, 1), lambda i: (i, 0)),
        ],
        out_shape=[
            jax.ShapeDtypeStruct((B, 2), f32),
            jax.ShapeDtypeStruct((B, 1), f32),
        ],
    )(x, g0, vecs, mats3)
    return y, jac


# speedup vs baseline: 2.1481x; 1.0327x over previous
import jax
import jax.numpy as jnp
from jax.experimental import pallas as pl

B = 16384
COND = 128
K = 10
BI = 5.0
BB = 512

V_B1T, V_B2T, V_B34T, V_B5T, V_KEEP, V_ADD = 0, 1, 2, 3, 4, 5
V_R0_1, V_C0_1, V_C1_1, V_C2_1 = 6, 7, 8, 9
V_R0_2, V_C0_2, V_C1_2, V_C2_2 = 10, 11, 12, 13
V_DBASE = 14
NVEC = 15

M_BD2, M_BD34, M_BD5 = 0, 1, 2
M_A0_1, M_A1_1, M_A2_1 = 3, 4, 5
M_A0_2, M_A1_2, M_A2_2 = 6, 7, 8
M_CW, M_CH, M_PD = 9, 10, 11
NMAT = 12


def _bdot(a, b):
    return jnp.dot(a, b, preferred_element_type=jnp.float32)


def _split(a):
    ahi = a.astype(jnp.bfloat16)
    alo = (a - ahi.astype(jnp.float32)).astype(jnp.bfloat16)
    return ahi, alo


def _dot3(a, w3):
    ahi, alo = _split(a)
    a3 = jnp.concatenate([ahi, alo, ahi], axis=1)
    return _bdot(a3, w3)


def _spline(raw, xcol, m3, dbase, li):
    liw = (li < K).astype(jnp.float32)
    lih = ((li >= K) & (li < 2 * K)).astype(jnp.float32)
    lid = ((li >= 2 * K) & (li < 3 * K - 1)).astype(jnp.float32)

    neg = jnp.float32(-1e30)
    mw = jnp.max(jnp.where(liw > 0, raw, neg), axis=1, keepdims=True)
    mh = jnp.max(jnp.where(lih > 0, raw, neg), axis=1, keepdims=True)
    ew = jnp.where(liw > 0, jnp.exp(raw - mw), 0.0)
    eh = jnp.where(lih > 0, jnp.exp(raw - mh), 0.0)
    sw = jnp.sum(ew, axis=1, keepdims=True)
    sh = jnp.sum(eh, axis=1, keepdims=True)
    cw = _dot3(ew, m3[M_CW])
    ch = _dot3(eh, m3[M_CH])
    w = cw * (2.0 * BI / sw) - BI
    h = ch * (2.0 * BI / sh) - BI
    sp = jnp.where(lid > 0, jax.nn.softplus(raw) + 0.001, 0.0)
    d = _dot3(sp, m3[M_PD]) + dbase

    cnt = jnp.sum(
        jnp.where((li <= K).astype(jnp.float32) * (w <= xcol) > 0, 1.0, 0.0),
        axis=1, keepdims=True)
    idx = jnp.clip(cnt, 1.0, float(K))
    lif = li.astype(jnp.float32)
    ohl = jnp.where(lif == idx - 1.0, 1.0, 0.0)
    ohr = jnp.where(lif == idx, 1.0, 0.0)

    def pick(arr, oh):
        return jnp.sum(arr * oh, axis=1, keepdims=True)

    wl, wr = pick(w, ohl), pick(w, ohr)
    hl, hr = pick(h, ohl), pick(h, ohr)
    dl, dr = pick(d, ohl), pick(d, ohr)

    tau = (xcol - wl) / (wr - wl)
    sk = (hr - hl) / (wr - wl)
    omt = 1.0 - tau
    v1 = sk * tau * tau + dl * tau * omt
    v2 = sk + (dr + dl - 2.0 * sk) * tau * omt
    y = hl + v1 / v2 * (hr - hl)
    d1 = sk * sk * (dr * tau * tau + 2.0 * sk * tau * omt + dl * omt * omt)
    pd = d1 / (v2 * v2)
    jac = jnp.abs(pd)
    return y, jac


def _g_kernel(c4_ref, pw1_ref, pt6_ref, vec_ref, m3_ref, g4_ref):
    vec = vec_ref[...]
    m3 = m3_ref

    def vrow(i):
        return vec[i:i + 1, :]

    c4 = c4_ref[...]
    h1 = jnp.maximum(_dot3(c4, pw1_ref[...]) + vrow(V_B1T), 0.0)
    h2 = jnp.maximum(_dot3(h1, m3[M_BD2]) + vrow(V_B2T), 0.0)
    m1 = jnp.maximum(_dot3(h2, m3[M_BD34]) + vrow(V_B34T), 0.0)
    m2 = jnp.maximum(_dot3(m1, m3[M_BD5]) + vrow(V_B5T), 0.0)
    g4_ref[...] = _dot3(m2, pt6_ref[...])


def _main_kernel(x_ref, g0_ref, vec_ref, m3_ref, y_ref, j_ref):
    vec = vec_ref[...]
    m3 = m3_ref
    li = jax.lax.broadcasted_iota(jnp.int32, (1, 128), 1)

    def vrow(i):
        return vec[i:i + 1, :]

    g = g0_ref[...] * vrow(V_KEEP) + vrow(V_ADD)

    x1 = x_ref[:, 0:1]
    x2 = x_ref[:, 1:2]

    def fstage(col, a0, a1, a2, r0, c0, c1, c2):
        hh = jnp.maximum(_dot3(g, m3[a0]) + col * vrow(r0) + vrow(c0), 0.0)
        hh = jnp.maximum(_dot3(hh, m3[a1]) + vrow(c1), 0.0)
        return _dot3(hh, m3[a2]) + vrow(c2)

    dbase = vrow(V_DBASE)
    raw2 = fstage(x1, M_A0_1, M_A1_1, M_A2_1, V_R0_1, V_C0_1, V_C1_1, V_C2_1)
    y2, jac2 = _spline(raw2, x2, m3, dbase, li)
    raw1 = fstage(y2, M_A0_2, M_A1_2, M_A2_2, V_R0_2, V_C0_2, V_C1_2, V_C2_2)
    y1, jac1 = _spline(raw1, x1, m3, dbase, li)

    y_ref[...] = jnp.concatenate([y1, y2], axis=1)
    j_ref[...] = jac1 * jac2


def _pad(a, shape):
    out = jnp.zeros(shape, jnp.float32)
    return out.at[tuple(slice(0, s) for s in a.shape)].set(a)


def kernel(x, c, params, index_p, index_v):
    f32 = jnp.float32
    (W1, b1), (W2, b2), (W3, b3) = params['fc_add']
    (W4, b4), (W5, b5), (W6, b6) = params['fc_min']
    null = params['null_token']
    num_nodes = COND // 2 + 1

    W34 = W3 @ W4
    b34 = b3 @ W4 + b4
    eye4 = jnp.eye(4, dtype=f32)
    pw1 = jnp.kron(eye4, W1)
    pt6 = jnp.kron(eye4, W6)
    bd2 = jnp.kron(eye4, W2)
    bd34 = jnp.kron(eye4, W34)
    bd5 = jnp.kron(eye4, W5)

    def tile4(v):
        return jnp.tile(v.reshape(1, 32), (1, 4)).reshape(128)

    nm1 = jax.nn.relu(null @ W4 + b4)
    nm2 = jax.nn.relu(nm1 @ W5 + b5)
    n0 = (nm2 @ W6 + b6)[0, 0]
    sv = jnp.sin(jnp.asarray(index_v, f32))
    gh1 = jax.nn.relu(sv * W1[0] + b1)
    gh2 = jax.nn.relu(gh1 @ W2 + b2)
    ga = gh2 @ W3 + b3
    gm1 = jax.nn.relu(ga @ W4 + b4)
    gm2 = jax.nn.relu(gm1 @ W5 + b5)
    gv = (gm2 @ W6 + b6)[0]

    ip = jnp.asarray(index_p, jnp.int32)
    keepv = jnp.ones((128,), f32).at[ip].set(0.0).at[ip + num_nodes - 1].set(0.0)
    addv = jnp.zeros((128,), f32).at[ip].set(n0).at[ip + num_nodes - 1].set(n0)
    addv = addv + b6[0] * keepv

    def fprep(layers):
        (U0, c0), (U1, c1), (U2, c2) = layers
        a0 = _pad(U0[1:1 + COND], (128, 128))
        r0 = _pad(U0[0:1], (1, 128))[0]
        cc0 = _pad((gv * U0[COND + 1] + c0)[None, :], (1, 128))[0]
        a1 = _pad(U1, (128, 128))
        cc1 = _pad(c1[None, :], (1, 128))[0]
        a2 = _pad(U2, (128, 128))
        cc2 = _pad(c2[None, :], (1, 128))[0]
        return a0, a1, a2, r0, cc0, cc1, cc2

    a0_1, a1_1, a2_1, r0_1, c0_1, c1_1, c2_1 = fprep(params['f1'])
    a0_2, a1_2, a2_2, r0_2, c0_2, c1_2, c2_2 = fprep(params['f2'])

    ii = jnp.arange(128)
    jj = jnp.arange(128)
    I, J = jnp.meshgrid(ii, jj, indexing='ij')
    cwm = ((I < K) & (J >= 1) & (J <= K) & (I <= J - 1)).astype(f32)
    chm = ((I >= K) & (I < 2 * K) & (J >= 1) & (J <= K)
           & (I - K <= J - 1)).astype(f32)
    pdm = ((I >= 2 * K) & (I < 3 * K - 1) & (J == I - 2 * K + 1)).astype(f32)
    dbase = jnp.zeros((128,), f32).at[0].set(1.0).at[K].set(1.0)

    vecs = jnp.stack([
        tile4(b1), tile4(b2), tile4(b34), tile4(b5), keepv, addv,
        r0_1, c0_1, c1_1, c2_1, r0_2, c0_2, c1_2, c2_2, dbase,
    ])
    matsarr = jnp.stack([
        bd2, bd34, bd5, a0_1, a1_1, a2_1, a0_2, a1_2, a2_2, cwm, chm, pdm,
    ])

    bf16 = jnp.bfloat16

    def w3(m):
        hi = m.astype(bf16)
        lo = (m - hi.astype(f32)).astype(bf16)
        return jnp.concatenate([hi, hi, lo], axis=0)

    mats3 = jax.vmap(w3)(matsarr)
    pw1s = w3(pw1)
    pt6s = w3(pt6)

    c4 = c.reshape(B * 32, 4)

    rep = lambda i: (0, 0)
    rep3 = lambda i: (0, 0, 0)
    g4 = pl.pallas_call(
        _g_kernel,
        grid=(B // BB,),
        in_specs=[
            pl.BlockSpec((BB * 32, 4), lambda i: (i, 0)),
            pl.BlockSpec((12, 128), rep),
            pl.BlockSpec((384, 4), rep),
            pl.BlockSpec((NVEC, 128), rep),
            pl.BlockSpec((NMAT, 384, 128), rep3),
        ],
        out_specs=pl.BlockSpec((BB * 32, 4), lambda i: (i, 0)),
        out_shape=jax.ShapeDtypeStruct((B * 32, 4), f32),
    )(c4, pw1s, pt6s, vecs, mats3)
    g0 = g4.reshape(B, 128)

    y, jac = pl.pallas_call(
        _main_kernel,
        grid=(B // BB,),
        in_specs=[
            pl.BlockSpec((BB, 2), lambda i: (i, 0)),
            pl.BlockSpec((BB, 128), lambda i: (i, 0)),
            pl.BlockSpec((NVEC, 128), rep),
            pl.BlockSpec((NMAT, 384, 128), rep3),
        ],
        out_specs=[
            pl.BlockSpec((BB, 2), lambda i: (i, 0)),
            pl.BlockSpec((BB, 1), lambda i: (i, 0)),
        ],
        out_shape=[
            jax.ShapeDtypeStruct((B, 2), f32),
            jax.ShapeDtypeStruct((B, 1), f32),
        ],
    )(x, g0, vecs, mats3)
    return y, jac


# speedup vs baseline: 4.4282x; 2.0615x over previous
import jax
import jax.numpy as jnp
from jax.experimental import pallas as pl

B = 16384
COND = 128
K = 10
BI = 5.0
BB = 256

V_KEEP, V_ADD = 0, 1
V_R0_1, V_C0_1, V_C1_1, V_C2_1 = 2, 3, 4, 5
V_R0_2, V_C0_2, V_C1_2, V_C2_2 = 6, 7, 8, 9
V_DBASE = 10
NVEC = 11

M_A0_1, M_A1_1, M_A2_1 = 0, 1, 2
M_A0_2, M_A1_2, M_A2_2 = 3, 4, 5
M_CW, M_CH, M_PD = 6, 7, 8
NMAT = 9

_bf16 = jnp.bfloat16


def _bdot(a, b):
    return jnp.dot(a, b, preferred_element_type=jnp.float32)


def _split(a):
    ahi = a.astype(_bf16)
    alo = (a - ahi.astype(jnp.float32)).astype(_bf16)
    return ahi, alo


def _dot3(a, w3):
    ahi, alo = _split(a)
    a3 = jnp.concatenate([ahi, alo, ahi], axis=1)
    return _bdot(a3, w3)


def _bat_dot(wa, rhs, out_dtype=jnp.float32):
    bb = rhs.shape[0]
    wab = jnp.broadcast_to(wa, (bb,) + wa.shape[1:])
    return jax.lax.dot_general(
        wab, rhs, (((2,), (1,)), ((0,), (0,))),
        preferred_element_type=out_dtype)


def _spline(raw, xcol, m3, dbase, li):
    liw = (li < K).astype(jnp.float32)
    lih = ((li >= K) & (li < 2 * K)).astype(jnp.float32)
    lid = ((li >= 2 * K) & (li < 3 * K - 1)).astype(jnp.float32)

    neg = jnp.float32(-1e30)
    mw = jnp.max(jnp.where(liw > 0, raw, neg), axis=1, keepdims=True)
    mh = jnp.max(jnp.where(lih > 0, raw, neg), axis=1, keepdims=True)
    ew = jnp.where(liw > 0, jnp.exp(raw - mw), 0.0)
    eh = jnp.where(lih > 0, jnp.exp(raw - mh), 0.0)
    sw = jnp.sum(ew, axis=1, keepdims=True)
    sh = jnp.sum(eh, axis=1, keepdims=True)
    cw = _dot3(ew, m3[M_CW])
    ch = _dot3(eh, m3[M_CH])
    w = cw * (2.0 * BI / sw) - BI
    h = ch * (2.0 * BI / sh) - BI
    sp = jnp.where(lid > 0, jax.nn.softplus(raw) + 0.001, 0.0)
    d = _dot3(sp, m3[M_PD]) + dbase

    cnt = jnp.sum(
        jnp.where((li <= K).astype(jnp.float32) * (w <= xcol) > 0, 1.0, 0.0),
        axis=1, keepdims=True)
    idx = jnp.clip(cnt, 1.0, float(K))
    lif = li.astype(jnp.float32)
    ohl = jnp.where(lif == idx - 1.0, 1.0, 0.0)
    ohr = jnp.where(lif == idx, 1.0, 0.0)

    def pick(arr, oh):
        return jnp.sum(arr * oh, axis=1, keepdims=True)

    wl, wr = pick(w, ohl), pick(w, ohr)
    hl, hr = pick(h, ohl), pick(h, ohr)
    dl, dr = pick(d, ohl), pick(d, ohr)

    tau = (xcol - wl) / (wr - wl)
    sk = (hr - hl) / (wr - wl)
    omt = 1.0 - tau
    v1 = sk * tau * tau + dl * tau * omt
    v2 = sk + (dr + dl - 2.0 * sk) * tau * omt
    y = hl + v1 / v2 * (hr - hl)
    d1 = sk * sk * (dr * tau * tau + 2.0 * sk * tau * omt + dl * omt * omt)
    pd = d1 / (v2 * v2)
    jac = jnp.abs(pd)
    return y, jac


def _fused_kernel(x_ref, c_ref, wa1_ref, wa2_ref, wa34_ref, wa5_ref,
                  waf_ref, vec_ref, m3_ref, y_ref, j_ref):
    vec = vec_ref[...]
    m3 = m3_ref
    li = jax.lax.broadcasted_iota(jnp.int32, (1, 128), 1)

    def vrow(i):
        return vec[i:i + 1, :]

    c = c_ref[...]
    chi = c.astype(_bf16)
    ones = jnp.ones((BB, 1, 128), _bf16)
    rhs1 = jnp.concatenate(
        [chi[:, None, :], chi[:, None, :], ones, ones], axis=1)
    h = jnp.maximum(_bat_dot(wa1_ref[...], rhs1), 0.0).astype(_bf16)

    def glayer(h, wa):
        rhs = jnp.concatenate([h, h, ones, ones], axis=1)
        return jnp.maximum(_bat_dot(wa, rhs), 0.0).astype(_bf16)

    h = glayer(h, wa2_ref[...])
    h = glayer(h, wa34_ref[...])
    h = glayer(h, wa5_ref[...])
    rhsf = jnp.concatenate([h, h], axis=1)
    g = _bat_dot(waf_ref[...], rhsf)[:, 0, :]
    g = g * vrow(V_KEEP) + vrow(V_ADD)

    x1 = x_ref[:, 0:1]
    x2 = x_ref[:, 1:2]

    def fstage(col, a0, a1, a2, r0, c0, c1, c2):
        hh = jnp.maximum(_dot3(g, m3[a0]) + col * vrow(r0) + vrow(c0), 0.0)
        hh = jnp.maximum(_dot3(hh, m3[a1]) + vrow(c1), 0.0)
        return _dot3(hh, m3[a2]) + vrow(c2)

    dbase = vrow(V_DBASE)
    raw2 = fstage(x1, M_A0_1, M_A1_1, M_A2_1, V_R0_1, V_C0_1, V_C1_1, V_C2_1)
    y2, jac2 = _spline(raw2, x2, m3, dbase, li)
    raw1 = fstage(y2, M_A0_2, M_A1_2, M_A2_2, V_R0_2, V_C0_2, V_C1_2, V_C2_2)
    y1, jac1 = _spline(raw1, x1, m3, dbase, li)

    y_ref[...] = jnp.concatenate([y1, y2], axis=1)
    j_ref[...] = jac1 * jac2


def _pad(a, shape):
    out = jnp.zeros(shape, jnp.float32)
    return out.at[tuple(slice(0, s) for s in a.shape)].set(a)


def _w3stack(m):
    hi = m.astype(_bf16)
    lo = (m - hi.astype(jnp.float32)).astype(_bf16)
    return jnp.concatenate([hi, hi, lo], axis=0)


def _hilo(v):
    hi = v.astype(_bf16)
    lo = (v - hi.astype(jnp.float32)).astype(_bf16)
    return hi, lo


def _walayer(W, b):
    whi, wlo = _hilo(W.T)
    bhi, blo = _hilo(b.reshape(32, 1))
    return jnp.concatenate([whi, wlo, bhi, blo], axis=1)[None]


def kernel(x, c, params, index_p, index_v):
    f32 = jnp.float32
    (W1, b1), (W2, b2), (W3, b3) = params['fc_add']
    (W4, b4), (W5, b5), (W6, b6) = params['fc_min']
    null = params['null_token']
    num_nodes = COND // 2 + 1

    W34 = W3 @ W4
    b34 = b3 @ W4 + b4

    w1hi, w1lo = _hilo(W1.reshape(32))
    b1hi, b1lo = _hilo(b1)
    wa1 = jnp.stack([w1hi, w1lo, b1hi, b1lo], axis=1)[None]
    wa2 = _walayer(W2, b2)
    wa34 = _walayer(W34, b34)
    wa5 = _walayer(W5, b5)
    w6hi, w6lo = _hilo(W6.reshape(32))
    waf = jnp.concatenate([w6hi, w6lo]).reshape(1, 1, 64)

    nm1 = jax.nn.relu(null @ W4 + b4)
    nm2 = jax.nn.relu(nm1 @ W5 + b5)
    n0 = (nm2 @ W6 + b6)[0, 0]
    sv = jnp.sin(jnp.asarray(index_v, f32))
    gh1 = jax.nn.relu(sv * W1[0] + b1)
    gh2 = jax.nn.relu(gh1 @ W2 + b2)
    ga = gh2 @ W3 + b3
    gm1 = jax.nn.relu(ga @ W4 + b4)
    gm2 = jax.nn.relu(gm1 @ W5 + b5)
    gv = (gm2 @ W6 + b6)[0]

    ip = jnp.asarray(index_p, jnp.int32)
    keepv = jnp.ones((128,), f32).at[ip].set(0.0).at[ip + num_nodes - 1].set(0.0)
    addv = jnp.zeros((128,), f32).at[ip].set(n0).at[ip + num_nodes - 1].set(n0)
    addv = addv + b6[0] * keepv

    def fprep(layers):
        (U0, c0), (U1, c1), (U2, c2) = layers
        a0 = _pad(U0[1:1 + COND], (128, 128))
        r0 = _pad(U0[0:1], (1, 128))[0]
        cc0 = _pad((gv * U0[COND + 1] + c0)[None, :], (1, 128))[0]
        a1 = _pad(U1, (128, 128))
        cc1 = _pad(c1[None, :], (1, 128))[0]
        a2 = _pad(U2, (128, 128))
        cc2 = _pad(c2[None, :], (1, 128))[0]
        return a0, a1, a2, r0, cc0, cc1, cc2

    a0_1, a1_1, a2_1, r0_1, c0_1, c1_1, c2_1 = fprep(params['f1'])
    a0_2, a1_2, a2_2, r0_2, c0_2, c1_2, c2_2 = fprep(params['f2'])

    ii = jnp.arange(128)
    jj = jnp.arange(128)
    I, J = jnp.meshgrid(ii, jj, indexing='ij')
    cwm = ((I < K) & (J >= 1) & (J <= K) & (I <= J - 1)).astype(f32)
    chm = ((I >= K) & (I < 2 * K) & (J >= 1) & (J <= K)
           & (I - K <= J - 1)).astype(f32)
    pdm = ((I >= 2 * K) & (I < 3 * K - 1) & (J == I - 2 * K + 1)).astype(f32)
    dbase = jnp.zeros((128,), f32).at[0].set(1.0).at[K].set(1.0)

    vecs = jnp.stack([
        keepv, addv,
        r0_1, c0_1, c1_1, c2_1, r0_2, c0_2, c1_2, c2_2, dbase,
    ])
    mats3 = jax.vmap(_w3stack)(jnp.stack([
        a0_1, a1_1, a2_1, a0_2, a1_2, a2_2, cwm, chm, pdm,
    ]))

    rep = lambda i: (0, 0)
    rep3 = lambda i: (0, 0, 0)
    y, jac = pl.pallas_call(
        _fused_kernel,
        grid=(B // BB,),
        in_specs=[
            pl.BlockSpec((BB, 2), lambda i: (i, 0)),
            pl.BlockSpec((BB, 128), lambda i: (i, 0)),
            pl.BlockSpec((1, 32, 4), rep3),
            pl.BlockSpec((1, 32, 66), rep3),
            pl.BlockSpec((1, 32, 66), rep3),
            pl.BlockSpec((1, 32, 66), rep3),
            pl.BlockSpec((1, 1, 64), rep3),
            pl.BlockSpec((NVEC, 128), rep),
            pl.BlockSpec((NMAT, 384, 128), rep3),
        ],
        out_specs=[
            pl.BlockSpec((BB, 2), lambda i: (i, 0)),
            pl.BlockSpec((BB, 1), lambda i: (i, 0)),
        ],
        out_shape=[
            jax.ShapeDtypeStruct((B, 2), f32),
            jax.ShapeDtypeStruct((B, 1), f32),
        ],
    )(x, c, wa1, wa2, wa34, wa5, waf, vecs, mats3)
    return y, jac


# speedup vs baseline: 4.4284x; 1.0001x over previous
import jax
import jax.numpy as jnp
import numpy as np
from jax.experimental import pallas as pl

B = 16384
COND = 128
K = 10
BI = 5.0
BB = 256

V_KEEP, V_ADD = 0, 1
V_R0_1, V_C0_1, V_C1_1, V_C2_1 = 2, 3, 4, 5
V_R0_2, V_C0_2, V_C1_2, V_C2_2 = 6, 7, 8, 9
V_DBASE = 10
NVEC = 11

M_A0_1, M_A1_1, M_A2_1 = 0, 1, 2
M_A0_2, M_A1_2, M_A2_2 = 3, 4, 5
M_CW, M_CH, M_PD = 6, 7, 8
NMAT = 9

_bf16 = jnp.bfloat16

_I, _J = np.meshgrid(np.arange(128), np.arange(128), indexing='ij')
_CWM = ((_I < K) & (_J >= 1) & (_J <= K) & (_I <= _J - 1))
_CHM = ((_I >= K) & (_I < 2 * K) & (_J >= 1) & (_J <= K)
        & (_I - K <= _J - 1))
_PDM = ((_I >= 2 * K) & (_I < 3 * K - 1) & (_J == _I - 2 * K + 1))


def _np_w3(m):
    m = m.astype(np.float32)
    return np.concatenate([m, m, np.zeros_like(m)], axis=0)


import ml_dtypes

_SPLINE_MATS3_NP = np.stack(
    [_np_w3(_CWM), _np_w3(_CHM), _np_w3(_PDM)]).astype(ml_dtypes.bfloat16)
_DBASE_NP = np.zeros((128,), np.float32)
_DBASE_NP[0] = 1.0
_DBASE_NP[K] = 1.0


def _bdot(a, b):
    return jnp.dot(a, b, preferred_element_type=jnp.float32)


def _split(a):
    ahi = a.astype(_bf16)
    alo = (a - ahi.astype(jnp.float32)).astype(_bf16)
    return ahi, alo


def _dot3(a, w3):
    ahi, alo = _split(a)
    a3 = jnp.concatenate([ahi, alo, ahi], axis=1)
    return _bdot(a3, w3)


def _bat_dot(wa, rhs, out_dtype=jnp.float32):
    bb = rhs.shape[0]
    wab = jnp.broadcast_to(wa, (bb,) + wa.shape[1:])
    return jax.lax.dot_general(
        wab, rhs, (((2,), (1,)), ((0,), (0,))),
        preferred_element_type=out_dtype)


def _spline(raw, xcol, m3, dbase, li):
    liw = (li < K).astype(jnp.float32)
    lih = ((li >= K) & (li < 2 * K)).astype(jnp.float32)
    lid = ((li >= 2 * K) & (li < 3 * K - 1)).astype(jnp.float32)

    neg = jnp.float32(-1e30)
    mw = jnp.max(jnp.where(liw > 0, raw, neg), axis=1, keepdims=True)
    mh = jnp.max(jnp.where(lih > 0, raw, neg), axis=1, keepdims=True)
    ew = jnp.where(liw > 0, jnp.exp(raw - mw), 0.0)
    eh = jnp.where(lih > 0, jnp.exp(raw - mh), 0.0)
    sw = jnp.sum(ew, axis=1, keepdims=True)
    sh = jnp.sum(eh, axis=1, keepdims=True)
    cw = _dot3(ew, m3[M_CW])
    ch = _dot3(eh, m3[M_CH])
    w = cw * (2.0 * BI / sw) - BI
    h = ch * (2.0 * BI / sh) - BI
    sp = jnp.where(lid > 0, jax.nn.softplus(raw) + 0.001, 0.0)
    d = _dot3(sp, m3[M_PD]) + dbase

    cnt = jnp.sum(
        jnp.where((li <= K).astype(jnp.float32) * (w <= xcol) > 0, 1.0, 0.0),
        axis=1, keepdims=True)
    idx = jnp.clip(cnt, 1.0, float(K))
    lif = li.astype(jnp.float32)
    ohl = jnp.where(lif == idx - 1.0, 1.0, 0.0)
    ohr = jnp.where(lif == idx, 1.0, 0.0)

    def pick(arr, oh):
        return jnp.sum(arr * oh, axis=1, keepdims=True)

    wl, wr = pick(w, ohl), pick(w, ohr)
    hl, hr = pick(h, ohl), pick(h, ohr)
    dl, dr = pick(d, ohl), pick(d, ohr)

    tau = (xcol - wl) / (wr - wl)
    sk = (hr - hl) / (wr - wl)
    omt = 1.0 - tau
    v1 = sk * tau * tau + dl * tau * omt
    v2 = sk + (dr + dl - 2.0 * sk) * tau * omt
    y = hl + v1 / v2 * (hr - hl)
    d1 = sk * sk * (dr * tau * tau + 2.0 * sk * tau * omt + dl * omt * omt)
    pd = d1 / (v2 * v2)
    jac = jnp.abs(pd)
    return y, jac


def _fused_kernel(x_ref, c_ref, wa1_ref, wa2_ref, wa34_ref, wa5_ref,
                  waf_ref, vec_ref, m3_ref, y_ref, j_ref):
    vec = vec_ref[...]
    m3 = m3_ref
    li = jax.lax.broadcasted_iota(jnp.int32, (1, 128), 1)

    def vrow(i):
        return vec[i:i + 1, :]

    c = c_ref[...]
    chi = c.astype(_bf16)
    ones = jnp.ones((BB, 1, 128), _bf16)
    rhs1 = jnp.concatenate(
        [chi[:, None, :], chi[:, None, :], ones, ones], axis=1)
    h = jnp.maximum(_bat_dot(wa1_ref[...], rhs1).astype(_bf16), 0)

    def glayer(h, wa):
        rhs = jnp.concatenate([h, h, ones, ones], axis=1)
        return jnp.maximum(_bat_dot(wa, rhs).astype(_bf16), 0)

    h = glayer(h, wa2_ref[...])
    h = glayer(h, wa34_ref[...])
    h = glayer(h, wa5_ref[...])
    rhsf = jnp.concatenate([h, h], axis=1)
    g = _bat_dot(waf_ref[...], rhsf)[:, 0, :]
    g = g * vrow(V_KEEP) + vrow(V_ADD)

    x1 = x_ref[:, 0:1]
    x2 = x_ref[:, 1:2]

    def fstage(col, a0, a1, a2, r0, c0, c1, c2):
        hh = jnp.maximum(_dot3(g, m3[a0]) + col * vrow(r0) + vrow(c0), 0.0)
        hh = jnp.maximum(_dot3(hh, m3[a1]) + vrow(c1), 0.0)
        return _dot3(hh, m3[a2]) + vrow(c2)

    dbase = vrow(V_DBASE)
    raw2 = fstage(x1, M_A0_1, M_A1_1, M_A2_1, V_R0_1, V_C0_1, V_C1_1, V_C2_1)
    y2, jac2 = _spline(raw2, x2, m3, dbase, li)
    raw1 = fstage(y2, M_A0_2, M_A1_2, M_A2_2, V_R0_2, V_C0_2, V_C1_2, V_C2_2)
    y1, jac1 = _spline(raw1, x1, m3, dbase, li)

    y_ref[...] = jnp.concatenate([y1, y2], axis=1)
    j_ref[...] = jac1 * jac2


def _pad(a, shape):
    out = jnp.zeros(shape, jnp.float32)
    return out.at[tuple(slice(0, s) for s in a.shape)].set(a)


def _w3stack(m):
    hi = m.astype(_bf16)
    lo = (m - hi.astype(jnp.float32)).astype(_bf16)
    return jnp.concatenate([hi, hi, lo], axis=0)


def _hilo(v):
    hi = v.astype(_bf16)
    lo = (v - hi.astype(jnp.float32)).astype(_bf16)
    return hi, lo


def _walayer(W, b):
    whi, wlo = _hilo(W.T)
    bhi, blo = _hilo(b.reshape(32, 1))
    return jnp.concatenate([whi, wlo, bhi, blo], axis=1)[None]


def kernel(x, c, params, index_p, index_v):
    f32 = jnp.float32
    (W1, b1), (W2, b2), (W3, b3) = params['fc_add']
    (W4, b4), (W5, b5), (W6, b6) = params['fc_min']
    null = params['null_token']
    num_nodes = COND // 2 + 1

    W34 = W3 @ W4
    b34 = b3 @ W4 + b4

    w1hi, w1lo = _hilo(W1.reshape(32))
    b1hi, b1lo = _hilo(b1)
    wa1 = jnp.stack([w1hi, w1lo, b1hi, b1lo], axis=1)[None]
    wa2 = _walayer(W2, b2)
    wa34 = _walayer(W34, b34)
    wa5 = _walayer(W5, b5)
    w6hi, w6lo = _hilo(W6.reshape(32))
    waf = jnp.concatenate([w6hi, w6lo]).reshape(1, 1, 64)

    nm1 = jax.nn.relu(null @ W4 + b4)
    nm2 = jax.nn.relu(nm1 @ W5 + b5)
    n0 = (nm2 @ W6 + b6)[0, 0]
    sv = jnp.sin(jnp.asarray(index_v, f32))
    gh1 = jax.nn.relu(sv * W1[0] + b1)
    gh2 = jax.nn.relu(gh1 @ W2 + b2)
    ga = gh2 @ W3 + b3
    gm1 = jax.nn.relu(ga @ W4 + b4)
    gm2 = jax.nn.relu(gm1 @ W5 + b5)
    gv = (gm2 @ W6 + b6)[0]

    ip = jnp.asarray(index_p, jnp.int32)
    keepv = jnp.ones((128,), f32).at[ip].set(0.0).at[ip + num_nodes - 1].set(0.0)
    addv = jnp.zeros((128,), f32).at[ip].set(n0).at[ip + num_nodes - 1].set(n0)
    addv = addv + b6[0] * keepv

    def fprep(layers):
        (U0, c0), (U1, c1), (U2, c2) = layers
        a0 = _pad(U0[1:1 + COND], (128, 128))
        r0 = _pad(U0[0:1], (1, 128))[0]
        cc0 = _pad((gv * U0[COND + 1] + c0)[None, :], (1, 128))[0]
        a1 = _pad(U1, (128, 128))
        cc1 = _pad(c1[None, :], (1, 128))[0]
        a2 = _pad(U2, (128, 128))
        cc2 = _pad(c2[None, :], (1, 128))[0]
        return a0, a1, a2, r0, cc0, cc1, cc2

    a0_1, a1_1, a2_1, r0_1, c0_1, c1_1, c2_1 = fprep(params['f1'])
    a0_2, a1_2, a2_2, r0_2, c0_2, c1_2, c2_2 = fprep(params['f2'])

    dbase = jnp.asarray(_DBASE_NP)

    vecs = jnp.stack([
        keepv, addv,
        r0_1, c0_1, c1_1, c2_1, r0_2, c0_2, c1_2, c2_2, dbase,
    ])
    mats3 = jnp.concatenate([
        jax.vmap(_w3stack)(jnp.stack(
            [a0_1, a1_1, a2_1, a0_2, a1_2, a2_2])),
        jnp.asarray(_SPLINE_MATS3_NP),
    ])

    rep = lambda i: (0, 0)
    rep3 = lambda i: (0, 0, 0)
    y, jac = pl.pallas_call(
        _fused_kernel,
        grid=(B // BB,),
        in_specs=[
            pl.BlockSpec((BB, 2), lambda i: (i, 0)),
            pl.BlockSpec((BB, 128), lambda i: (i, 0)),
            pl.BlockSpec((1, 32, 4), rep3),
            pl.BlockSpec((1, 32, 66), rep3),
            pl.BlockSpec((1, 32, 66), rep3),
            pl.BlockSpec((1, 32, 66), rep3),
            pl.BlockSpec((1, 1, 64), rep3),
            pl.BlockSpec((NVEC, 128), rep),
            pl.BlockSpec((NMAT, 384, 128), rep3),
        ],
        out_specs=[
            pl.BlockSpec((BB, 2), lambda i: (i, 0)),
            pl.BlockSpec((BB, 1), lambda i: (i, 0)),
        ],
        out_shape=[
            jax.ShapeDtypeStruct((B, 2), f32),
            jax.ShapeDtypeStruct((B, 1), f32),
        ],
    )(x, c, wa1, wa2, wa34, wa5, waf, vecs, mats3)
    return y, jac


# speedup vs baseline: 4.7780x; 1.0789x over previous
import jax
import jax.numpy as jnp
import numpy as np
from jax.experimental import pallas as pl

B = 16384
COND = 128
K = 10
BI = 5.0
BB = 1024

V_KEEP, V_ADD = 0, 1
V_R0_1, V_C0_1, V_C1_1, V_C2_1 = 2, 3, 4, 5
V_R0_2, V_C0_2, V_C1_2, V_C2_2 = 6, 7, 8, 9
V_DBASE = 10
NVEC = 11

M_A0_1, M_A1_1, M_A2_1 = 0, 1, 2
M_A0_2, M_A1_2, M_A2_2 = 3, 4, 5
M_CW, M_CH, M_PD = 6, 7, 8
NMAT = 9

_bf16 = jnp.bfloat16

_I, _J = np.meshgrid(np.arange(128), np.arange(128), indexing='ij')
_CWM = ((_I < K) & (_J >= 1) & (_J <= K) & (_I <= _J - 1))
_CHM = ((_I >= K) & (_I < 2 * K) & (_J >= 1) & (_J <= K)
        & (_I - K <= _J - 1))
_PDM = ((_I >= 2 * K) & (_I < 3 * K - 1) & (_J == _I - 2 * K + 1))


def _np_w3(m):
    m = m.astype(np.float32)
    return np.concatenate([m, m, np.zeros_like(m)], axis=0)


import ml_dtypes

_SPLINE_MATS3_NP = np.stack(
    [_np_w3(_CWM), _np_w3(_CHM), _np_w3(_PDM)]).astype(ml_dtypes.bfloat16)
_DBASE_NP = np.zeros((128,), np.float32)
_DBASE_NP[0] = 1.0
_DBASE_NP[K] = 1.0


def _bdot(a, b):
    return jnp.dot(a, b, preferred_element_type=jnp.float32)


def _split(a):
    ahi = a.astype(_bf16)
    alo = (a - ahi.astype(jnp.float32)).astype(_bf16)
    return ahi, alo


def _dot3(a, w3):
    ahi, alo = _split(a)
    a3 = jnp.concatenate([ahi, alo, ahi], axis=1)
    return _bdot(a3, w3)


def _bat_dot(wa, rhs, out_dtype=jnp.float32):
    bb = rhs.shape[0]
    wab = jnp.broadcast_to(wa, (bb,) + wa.shape[1:])
    return jax.lax.dot_general(
        wab, rhs, (((2,), (1,)), ((0,), (0,))),
        preferred_element_type=out_dtype)


def _spline(raw, xcol, m3, dbase, li):
    liw = (li < K).astype(jnp.float32)
    lih = ((li >= K) & (li < 2 * K)).astype(jnp.float32)
    lid = ((li >= 2 * K) & (li < 3 * K - 1)).astype(jnp.float32)

    neg = jnp.float32(-1e30)
    mw = jnp.max(jnp.where(liw > 0, raw, neg), axis=1, keepdims=True)
    mh = jnp.max(jnp.where(lih > 0, raw, neg), axis=1, keepdims=True)
    ew = jnp.where(liw > 0, jnp.exp(raw - mw), 0.0)
    eh = jnp.where(lih > 0, jnp.exp(raw - mh), 0.0)
    sw = jnp.sum(ew, axis=1, keepdims=True)
    sh = jnp.sum(eh, axis=1, keepdims=True)
    cw = _dot3(ew, m3[M_CW])
    ch = _dot3(eh, m3[M_CH])
    w = cw * (2.0 * BI / sw) - BI
    h = ch * (2.0 * BI / sh) - BI
    sp = jnp.where(lid > 0, jax.nn.softplus(raw) + 0.001, 0.0)
    d = _dot3(sp, m3[M_PD]) + dbase

    cnt = jnp.sum(
        jnp.where((li <= K).astype(jnp.float32) * (w <= xcol) > 0, 1.0, 0.0),
        axis=1, keepdims=True)
    idx = jnp.clip(cnt, 1.0, float(K))
    lif = li.astype(jnp.float32)
    ohl = jnp.where(lif == idx - 1.0, 1.0, 0.0)
    ohr = jnp.where(lif == idx, 1.0, 0.0)

    def pick(arr, oh):
        return jnp.sum(arr * oh, axis=1, keepdims=True)

    wl, wr = pick(w, ohl), pick(w, ohr)
    hl, hr = pick(h, ohl), pick(h, ohr)
    dl, dr = pick(d, ohl), pick(d, ohr)

    tau = (xcol - wl) / (wr - wl)
    sk = (hr - hl) / (wr - wl)
    omt = 1.0 - tau
    v1 = sk * tau * tau + dl * tau * omt
    v2 = sk + (dr + dl - 2.0 * sk) * tau * omt
    y = hl + v1 / v2 * (hr - hl)
    d1 = sk * sk * (dr * tau * tau + 2.0 * sk * tau * omt + dl * omt * omt)
    pd = d1 / (v2 * v2)
    jac = jnp.abs(pd)
    return y, jac


def _fused_kernel(x_ref, c_ref, wa1_ref, wa2_ref, wa34_ref, wa5_ref,
                  waf_ref, vec_ref, m3_ref, y_ref, j_ref):
    vec = vec_ref[...]
    m3 = m3_ref
    li = jax.lax.broadcasted_iota(jnp.int32, (1, 128), 1)

    def vrow(i):
        return vec[i:i + 1, :]

    c = c_ref[...]
    chi = c.astype(_bf16)
    ones = jnp.ones((BB, 1, 128), _bf16)
    rhs1 = jnp.concatenate(
        [chi[:, None, :], chi[:, None, :], ones, ones], axis=1)
    h = jnp.maximum(_bat_dot(wa1_ref[...], rhs1).astype(_bf16), 0)

    def glayer(h, wa):
        rhs = jnp.concatenate([h, h, ones, ones], axis=1)
        return jnp.maximum(_bat_dot(wa, rhs).astype(_bf16), 0)

    h = glayer(h, wa2_ref[...])
    h = glayer(h, wa34_ref[...])
    h = glayer(h, wa5_ref[...])
    rhsf = jnp.concatenate([h, h], axis=1)
    g = _bat_dot(waf_ref[...], rhsf)[:, 0, :]
    g = g * vrow(V_KEEP) + vrow(V_ADD)

    x1 = x_ref[:, 0:1]
    x2 = x_ref[:, 1:2]

    def fstage(col, a0, a1, a2, r0, c0, c1, c2):
        hh = jnp.maximum(_dot3(g, m3[a0]) + col * vrow(r0) + vrow(c0), 0.0)
        hh = jnp.maximum(_dot3(hh, m3[a1]) + vrow(c1), 0.0)
        return _dot3(hh, m3[a2]) + vrow(c2)

    dbase = vrow(V_DBASE)
    raw2 = fstage(x1, M_A0_1, M_A1_1, M_A2_1, V_R0_1, V_C0_1, V_C1_1, V_C2_1)
    y2, jac2 = _spline(raw2, x2, m3, dbase, li)
    raw1 = fstage(y2, M_A0_2, M_A1_2, M_A2_2, V_R0_2, V_C0_2, V_C1_2, V_C2_2)
    y1, jac1 = _spline(raw1, x1, m3, dbase, li)

    y_ref[...] = jnp.concatenate([y1, y2], axis=1)
    j_ref[...] = jac1 * jac2


def _pad(a, shape):
    out = jnp.zeros(shape, jnp.float32)
    return out.at[tuple(slice(0, s) for s in a.shape)].set(a)


def _w3stack(m):
    hi = m.astype(_bf16)
    lo = (m - hi.astype(jnp.float32)).astype(_bf16)
    return jnp.concatenate([hi, hi, lo], axis=0)


def _hilo(v):
    hi = v.astype(_bf16)
    lo = (v - hi.astype(jnp.float32)).astype(_bf16)
    return hi, lo


def _walayer(W, b):
    whi, wlo = _hilo(W.T)
    bhi, blo = _hilo(b.reshape(32, 1))
    return jnp.concatenate([whi, wlo, bhi, blo], axis=1)[None]


def kernel(x, c, params, index_p, index_v):
    f32 = jnp.float32
    (W1, b1), (W2, b2), (W3, b3) = params['fc_add']
    (W4, b4), (W5, b5), (W6, b6) = params['fc_min']
    null = params['null_token']
    num_nodes = COND // 2 + 1

    W34 = W3 @ W4
    b34 = b3 @ W4 + b4

    w1hi, w1lo = _hilo(W1.reshape(32))
    b1hi, b1lo = _hilo(b1)
    wa1 = jnp.stack([w1hi, w1lo, b1hi, b1lo], axis=1)[None]
    wa2 = _walayer(W2, b2)
    wa34 = _walayer(W34, b34)
    wa5 = _walayer(W5, b5)
    w6hi, w6lo = _hilo(W6.reshape(32))
    waf = jnp.concatenate([w6hi, w6lo]).reshape(1, 1, 64)

    nm1 = jax.nn.relu(null @ W4 + b4)
    nm2 = jax.nn.relu(nm1 @ W5 + b5)
    n0 = (nm2 @ W6 + b6)[0, 0]
    sv = jnp.sin(jnp.asarray(index_v, f32))
    gh1 = jax.nn.relu(sv * W1[0] + b1)
    gh2 = jax.nn.relu(gh1 @ W2 + b2)
    ga = gh2 @ W3 + b3
    gm1 = jax.nn.relu(ga @ W4 + b4)
    gm2 = jax.nn.relu(gm1 @ W5 + b5)
    gv = (gm2 @ W6 + b6)[0]

    ip = jnp.asarray(index_p, jnp.int32)
    keepv = jnp.ones((128,), f32).at[ip].set(0.0).at[ip + num_nodes - 1].set(0.0)
    addv = jnp.zeros((128,), f32).at[ip].set(n0).at[ip + num_nodes - 1].set(n0)
    addv = addv + b6[0] * keepv

    def fprep(layers):
        (U0, c0), (U1, c1), (U2, c2) = layers
        a0 = _pad(U0[1:1 + COND], (128, 128))
        r0 = _pad(U0[0:1], (1, 128))[0]
        cc0 = _pad((gv * U0[COND + 1] + c0)[None, :], (1, 128))[0]
        a1 = _pad(U1, (128, 128))
        cc1 = _pad(c1[None, :], (1, 128))[0]
        a2 = _pad(U2, (128, 128))
        cc2 = _pad(c2[None, :], (1, 128))[0]
        return a0, a1, a2, r0, cc0, cc1, cc2

    a0_1, a1_1, a2_1, r0_1, c0_1, c1_1, c2_1 = fprep(params['f1'])
    a0_2, a1_2, a2_2, r0_2, c0_2, c1_2, c2_2 = fprep(params['f2'])

    dbase = jnp.asarray(_DBASE_NP)

    vecs = jnp.stack([
        keepv, addv,
        r0_1, c0_1, c1_1, c2_1, r0_2, c0_2, c1_2, c2_2, dbase,
    ])
    mats3 = jnp.concatenate([
        jax.vmap(_w3stack)(jnp.stack(
            [a0_1, a1_1, a2_1, a0_2, a1_2, a2_2])),
        jnp.asarray(_SPLINE_MATS3_NP),
    ])

    rep = lambda i: (0, 0)
    rep3 = lambda i: (0, 0, 0)
    y, jac = pl.pallas_call(
        _fused_kernel,
        grid=(B // BB,),
        in_specs=[
            pl.BlockSpec((BB, 2), lambda i: (i, 0)),
            pl.BlockSpec((BB, 128), lambda i: (i, 0)),
            pl.BlockSpec((1, 32, 4), rep3),
            pl.BlockSpec((1, 32, 66), rep3),
            pl.BlockSpec((1, 32, 66), rep3),
            pl.BlockSpec((1, 32, 66), rep3),
            pl.BlockSpec((1, 1, 64), rep3),
            pl.BlockSpec((NVEC, 128), rep),
            pl.BlockSpec((NMAT, 384, 128), rep3),
        ],
        out_specs=[
            pl.BlockSpec((BB, 2), lambda i: (i, 0)),
            pl.BlockSpec((BB, 1), lambda i: (i, 0)),
        ],
        out_shape=[
            jax.ShapeDtypeStruct((B, 2), f32),
            jax.ShapeDtypeStruct((B, 1), f32),
        ],
    )(x, c, wa1, wa2, wa34, wa5, waf, vecs, mats3)
    return y, jac


# speedup vs baseline: 4.8470x; 1.0144x over previous
import jax
import jax.numpy as jnp
import numpy as np
from jax.experimental import pallas as pl

B = 16384
COND = 128
K = 10
BI = 5.0
BB = 1024

V_KEEP, V_ADD = 0, 1
V_R0_1, V_C0_1, V_C1_1, V_C2_1 = 2, 3, 4, 5
V_R0_2, V_C0_2, V_C1_2, V_C2_2 = 6, 7, 8, 9
V_DBASE = 10
NVEC = 11

M_A0_1, M_A1_1, M_A2_1 = 0, 1, 2
M_A0_2, M_A1_2, M_A2_2 = 3, 4, 5
M_CW, M_CH, M_PD = 6, 7, 8
NMAT = 9

_bf16 = jnp.bfloat16

_I, _J = np.meshgrid(np.arange(128), np.arange(128), indexing='ij')
_CWM = ((_I < K) & (_J >= 1) & (_J <= K) & (_I <= _J - 1))
_CHM = ((_I >= K) & (_I < 2 * K) & (_J >= 1) & (_J <= K)
        & (_I - K <= _J - 1))
_PDM = ((_I >= 2 * K) & (_I < 3 * K - 1) & (_J == _I - 2 * K + 1))


import ml_dtypes


def _np_w2(m):
    m = m.astype(np.float32)
    return np.concatenate([m, m], axis=0)


_SPLINE_MATS3_NP = np.stack(
    [_np_w2(_CWM), _np_w2(_CHM), _np_w2(_PDM)]).astype(ml_dtypes.bfloat16)
_DBASE_NP = np.zeros((128,), np.float32)
_DBASE_NP[0] = 1.0
_DBASE_NP[K] = 1.0


def _bdot(a, b):
    return jnp.dot(a, b, preferred_element_type=jnp.float32)


def _split(a):
    ahi = a.astype(_bf16)
    alo = (a - ahi.astype(jnp.float32)).astype(_bf16)
    return ahi, alo


def _dot2a(a, w2):
    ahi = a.astype(_bf16)
    a2 = jnp.concatenate([ahi, ahi], axis=1)
    return _bdot(a2, w2)


def _dot2x(a, w2):
    ahi, alo = _split(a)
    a2 = jnp.concatenate([ahi, alo], axis=1)
    return _bdot(a2, w2)


def _bat_dot(wa, rhs, out_dtype=jnp.float32):
    bb = rhs.shape[0]
    wab = jnp.broadcast_to(wa, (bb,) + wa.shape[1:])
    return jax.lax.dot_general(
        wab, rhs, (((2,), (1,)), ((0,), (0,))),
        preferred_element_type=out_dtype)


def _spline(raw, xcol, m3, dbase, li):
    liw = (li < K).astype(jnp.float32)
    lih = ((li >= K) & (li < 2 * K)).astype(jnp.float32)
    lid = ((li >= 2 * K) & (li < 3 * K - 1)).astype(jnp.float32)

    neg = jnp.float32(-1e30)
    mw = jnp.max(jnp.where(liw > 0, raw, neg), axis=1, keepdims=True)
    mh = jnp.max(jnp.where(lih > 0, raw, neg), axis=1, keepdims=True)
    ew = jnp.where(liw > 0, jnp.exp(raw - mw), 0.0)
    eh = jnp.where(lih > 0, jnp.exp(raw - mh), 0.0)
    sw = jnp.sum(ew, axis=1, keepdims=True)
    sh = jnp.sum(eh, axis=1, keepdims=True)
    cw = _dot2x(ew, m3[M_CW])
    ch = _dot2x(eh, m3[M_CH])
    w = cw * (2.0 * BI / sw) - BI
    h = ch * (2.0 * BI / sh) - BI
    sp = jnp.where(lid > 0, jax.nn.softplus(raw) + 0.001, 0.0)
    d = _dot2x(sp, m3[M_PD]) + dbase

    cnt = jnp.sum(
        jnp.where((li <= K).astype(jnp.float32) * (w <= xcol) > 0, 1.0, 0.0),
        axis=1, keepdims=True)
    idx = jnp.clip(cnt, 1.0, float(K))
    lif = li.astype(jnp.float32)
    ohl = jnp.where(lif == idx - 1.0, 1.0, 0.0)
    ohr = jnp.where(lif == idx, 1.0, 0.0)

    def pick(arr, oh):
        return jnp.sum(arr * oh, axis=1, keepdims=True)

    wl, wr = pick(w, ohl), pick(w, ohr)
    hl, hr = pick(h, ohl), pick(h, ohr)
    dl, dr = pick(d, ohl), pick(d, ohr)

    tau = (xcol - wl) / (wr - wl)
    sk = (hr - hl) / (wr - wl)
    omt = 1.0 - tau
    v1 = sk * tau * tau + dl * tau * omt
    v2 = sk + (dr + dl - 2.0 * sk) * tau * omt
    y = hl + v1 / v2 * (hr - hl)
    d1 = sk * sk * (dr * tau * tau + 2.0 * sk * tau * omt + dl * omt * omt)
    pd = d1 / (v2 * v2)
    jac = jnp.abs(pd)
    return y, jac


def _fused_kernel(x_ref, c_ref, wa1_ref, wa2_ref, wa34_ref, wa5_ref,
                  waf_ref, vec_ref, m3_ref, y_ref, j_ref):
    vec = vec_ref[...]
    m3 = m3_ref
    li = jax.lax.broadcasted_iota(jnp.int32, (1, 128), 1)

    def vrow(i):
        return vec[i:i + 1, :]

    c = c_ref[...]
    chi = c.astype(_bf16)
    ones = jnp.ones((BB, 1, 128), _bf16)
    rhs1 = jnp.concatenate(
        [chi[:, None, :], chi[:, None, :], ones, ones], axis=1)
    h = jnp.maximum(_bat_dot(wa1_ref[...], rhs1).astype(_bf16), 0)

    def glayer(h, wa):
        rhs = jnp.concatenate([h, h, ones, ones], axis=1)
        return jnp.maximum(_bat_dot(wa, rhs).astype(_bf16), 0)

    h = glayer(h, wa2_ref[...])
    h = glayer(h, wa34_ref[...])
    h = glayer(h, wa5_ref[...])
    rhsf = jnp.concatenate([h, h], axis=1)
    g = _bat_dot(waf_ref[...], rhsf)[:, 0, :]
    g = g * vrow(V_KEEP) + vrow(V_ADD)

    x1 = x_ref[:, 0:1]
    x2 = x_ref[:, 1:2]

    def fstage(col, a0, a1, a2, r0, c0, c1, c2):
        hh = jnp.maximum(_dot2a(g, m3[a0]) + col * vrow(r0) + vrow(c0), 0.0)
        hh = jnp.maximum(_dot2a(hh, m3[a1]) + vrow(c1), 0.0)
        return _dot2a(hh, m3[a2]) + vrow(c2)

    dbase = vrow(V_DBASE)
    raw2 = fstage(x1, M_A0_1, M_A1_1, M_A2_1, V_R0_1, V_C0_1, V_C1_1, V_C2_1)
    y2, jac2 = _spline(raw2, x2, m3, dbase, li)
    raw1 = fstage(y2, M_A0_2, M_A1_2, M_A2_2, V_R0_2, V_C0_2, V_C1_2, V_C2_2)
    y1, jac1 = _spline(raw1, x1, m3, dbase, li)

    y_ref[...] = jnp.concatenate([y1, y2], axis=1)
    j_ref[...] = jac1 * jac2


def _pad(a, shape):
    out = jnp.zeros(shape, jnp.float32)
    return out.at[tuple(slice(0, s) for s in a.shape)].set(a)


def _w2stack(m):
    hi = m.astype(_bf16)
    lo = (m - hi.astype(jnp.float32)).astype(_bf16)
    return jnp.concatenate([hi, lo], axis=0)


def _hilo(v):
    hi = v.astype(_bf16)
    lo = (v - hi.astype(jnp.float32)).astype(_bf16)
    return hi, lo


def _walayer(W, b):
    whi, wlo = _hilo(W.T)
    bhi, blo = _hilo(b.reshape(32, 1))
    return jnp.concatenate([whi, wlo, bhi, blo], axis=1)[None]


def kernel(x, c, params, index_p, index_v):
    f32 = jnp.float32
    (W1, b1), (W2, b2), (W3, b3) = params['fc_add']
    (W4, b4), (W5, b5), (W6, b6) = params['fc_min']
    null = params['null_token']
    num_nodes = COND // 2 + 1

    W34 = W3 @ W4
    b34 = b3 @ W4 + b4

    w1hi, w1lo = _hilo(W1.reshape(32))
    b1hi, b1lo = _hilo(b1)
    wa1 = jnp.stack([w1hi, w1lo, b1hi, b1lo], axis=1)[None]
    wa2 = _walayer(W2, b2)
    wa34 = _walayer(W34, b34)
    wa5 = _walayer(W5, b5)
    w6hi, w6lo = _hilo(W6.reshape(32))
    waf = jnp.concatenate([w6hi, w6lo]).reshape(1, 1, 64)

    nm1 = jax.nn.relu(null @ W4 + b4)
    nm2 = jax.nn.relu(nm1 @ W5 + b5)
    n0 = (nm2 @ W6 + b6)[0, 0]
    sv = jnp.sin(jnp.asarray(index_v, f32))
    gh1 = jax.nn.relu(sv * W1[0] + b1)
    gh2 = jax.nn.relu(gh1 @ W2 + b2)
    ga = gh2 @ W3 + b3
    gm1 = jax.nn.relu(ga @ W4 + b4)
    gm2 = jax.nn.relu(gm1 @ W5 + b5)
    gv = (gm2 @ W6 + b6)[0]

    ip = jnp.asarray(index_p, jnp.int32)
    keepv = jnp.ones((128,), f32).at[ip].set(0.0).at[ip + num_nodes - 1].set(0.0)
    addv = jnp.zeros((128,), f32).at[ip].set(n0).at[ip + num_nodes - 1].set(n0)
    addv = addv + b6[0] * keepv

    def fprep(layers):
        (U0, c0), (U1, c1), (U2, c2) = layers
        a0 = _pad(U0[1:1 + COND], (128, 128))
        r0 = _pad(U0[0:1], (1, 128))[0]
        cc0 = _pad((gv * U0[COND + 1] + c0)[None, :], (1, 128))[0]
        a1 = _pad(U1, (128, 128))
        cc1 = _pad(c1[None, :], (1, 128))[0]
        a2 = _pad(U2, (128, 128))
        cc2 = _pad(c2[None, :], (1, 128))[0]
        return a0, a1, a2, r0, cc0, cc1, cc2

    a0_1, a1_1, a2_1, r0_1, c0_1, c1_1, c2_1 = fprep(params['f1'])
    a0_2, a1_2, a2_2, r0_2, c0_2, c1_2, c2_2 = fprep(params['f2'])

    dbase = jnp.asarray(_DBASE_NP)

    vecs = jnp.stack([
        keepv, addv,
        r0_1, c0_1, c1_1, c2_1, r0_2, c0_2, c1_2, c2_2, dbase,
    ])
    mats3 = jnp.concatenate([
        jax.vmap(_w2stack)(jnp.stack(
            [a0_1, a1_1, a2_1, a0_2, a1_2, a2_2])),
        jnp.asarray(_SPLINE_MATS3_NP),
    ])

    rep = lambda i: (0, 0)
    rep3 = lambda i: (0, 0, 0)
    y, jac = pl.pallas_call(
        _fused_kernel,
        grid=(B // BB,),
        in_specs=[
            pl.BlockSpec((BB, 2), lambda i: (i, 0)),
            pl.BlockSpec((BB, 128), lambda i: (i, 0)),
            pl.BlockSpec((1, 32, 4), rep3),
            pl.BlockSpec((1, 32, 66), rep3),
            pl.BlockSpec((1, 32, 66), rep3),
            pl.BlockSpec((1, 32, 66), rep3),
            pl.BlockSpec((1, 1, 64), rep3),
            pl.BlockSpec((NVEC, 128), rep),
            pl.BlockSpec((NMAT, 256, 128), rep3),
        ],
        out_specs=[
            pl.BlockSpec((BB, 2), lambda i: (i, 0)),
            pl.BlockSpec((BB, 1), lambda i: (i, 0)),
        ],
        out_shape=[
            jax.ShapeDtypeStruct((B, 2), f32),
            jax.ShapeDtypeStruct((B, 1), f32),
        ],
    )(x, c, wa1, wa2, wa34, wa5, waf, vecs, mats3)
    return y, jac
